# Initial kernel scaffold; baseline (speedup 1.0000x reference)
import jax
import jax.numpy as jnp
from jax.experimental import pallas as pl


def kernel(x, edge_index, W1, a_src1, a_dst1, b1, W2, a_src2, a_dst2, b2):
    raise NotImplementedError("write your pallas kernel here")


# speedup vs baseline: 4.9918x; 4.9918x over previous
import functools

import jax
import jax.numpy as jnp
from jax import lax
from jax.experimental import pallas as pl
from jax.experimental.pallas import tpu as pltpu
from jax.experimental.pallas import tpu_sc as plsc

_N = 10000
_E = 320000
_DIN = 128
_HID = 64
_HEADS = 8
_NCLS = 40

_NC = 2
_NS = 16
_NPAD = 10240
_EPAD = 327680
_RH = 5120
_RACC = 5376
_ROOB = 5200
_BCH = 8

_CA = 64
_CHA_TOT = _EPAD // _CA
_CHA_W = _CHA_TOT // (_NC * _NS)
_CWA = _BCH * _CA * 16

_CB = 128
_CHB_TOT = _EPAD // _CB
_CHB_W = _CHB_TOT // (_NC * _NS)
_CHB_S = _CHB_TOT // _NS
_CWB = _BCH * _CB * 16

_ZRA = _RACC // _NS
_WRH = _RH // _NS

_f32 = jnp.float32


@functools.cache
def _sc_mesh():
    return plsc.VectorSubcoreMesh(core_axis_name="c", subcore_axis_name="s")


def _zero16():
    return jnp.zeros((16,), _f32)


def _fill_zb(zb):
    def _zrow(i, _):
        for k in range(8):
            zb[i, pl.ds(k * 16, 16)] = _zero16()
        return 0
    lax.fori_loop(0, 16, _zrow, 0)


def _zero_acc(zb, acc_sh, s):
    def _zcp(t, _):
        pltpu.sync_copy(zb, acc_sh.at[pl.ds(s * _ZRA + t * 16, 16), :])
        return 0
    lax.fori_loop(0, _ZRA // 16, _zcp, 0)


def _clamp_idx(didx, didx2, nrows, base, groups=8):
    def _row(r, _):
        for k in range(groups):
            sl = pl.ds(k * 16, 16)
            v = didx[r, sl] - base
            oob = (v < 0) | (v >= _RH)
            didx2[r, sl] = jnp.where(oob, _ROOB, v)
        return 0
    lax.fori_loop(0, nrows, _row, 0)


def _attn_body(src2, dst2, als, ald, g16, w_out, den_out,
               sidx, didx, didx2, asv, adv, wflat, wpad, gv, zb, den_sh, sem):
    c = lax.axis_index("c")
    s = lax.axis_index("s")
    wid = s * _NC + c
    pltpu.sync_copy(g16, gv)
    _fill_zb(zb)

    def _zpad(j, _):
        for k in range(8):
            wpad[j, pl.ds(k * 16, 16)] = _zero16()
        return 0
    lax.fori_loop(0, _CA, _zpad, 0)

    g = gv[...]

    for half in range(2):
        base = half * _RH
        _zero_acc(zb, den_sh, s)
        plsc.subcore_barrier()

        def _batch(t, _):
            ch0 = wid * _CHA_W + t * _BCH
            pltpu.sync_copy(src2.at[pl.ds(ch0, _BCH), :], sidx)
            pltpu.sync_copy(dst2.at[pl.ds(ch0, _BCH), :], didx)
            _clamp_idx(didx, didx2, _BCH, base, groups=_CA // 16)

            def _chunk(b, __):
                pltpu.async_copy(als.at[sidx.at[b]], asv, sem).wait()
                pltpu.async_copy(ald.at[didx.at[b]], adv, sem).wait()

                def _edge(j, ___):
                    v = asv[j, pl.ds(0, 16)] + adv[j, pl.ds(0, 16)]
                    e = jnp.where(v >= 0.0, v, 0.2 * v)
                    w = jnp.exp(e - g)
                    wflat[pl.ds((b * _CA + j) * 16, 16)] = w
                    wpad[j, pl.ds(0, 16)] = w
                    return 0
                lax.fori_loop(0, _CA, _edge, 0)
                pltpu.sync_copy(wpad, den_sh.at[didx2.at[b]], add=True)
                return 0
            lax.fori_loop(0, _BCH, _chunk, 0)
            if half == 0:
                pltpu.sync_copy(wflat, w_out.at[pl.ds(ch0 * _CA * 16, _CWA)])
            return 0
        lax.fori_loop(0, _CHA_W // _BCH, _batch, 0)

        plsc.subcore_barrier()
        pltpu.sync_copy(den_sh.at[pl.ds(s * _WRH, _WRH), :],
                        den_out.at[c, pl.ds(base + s * _WRH, _WRH), :])
        plsc.subcore_barrier()


@functools.cache
def _attn_kernel():
    return pl.kernel(
        _attn_body,
        mesh=_sc_mesh(),
        out_type=[jax.ShapeDtypeStruct((_EPAD * 16,), _f32),
                  jax.ShapeDtypeStruct((_NC, _NPAD, 128), _f32)],
        scratch_types=[pltpu.VMEM((_BCH, _CA), jnp.int32),
                       pltpu.VMEM((_BCH, _CA), jnp.int32),
                       pltpu.VMEM((_BCH, _CA), jnp.int32),
                       pltpu.VMEM((_CA, 128), _f32),
                       pltpu.VMEM((_CA, 128), _f32),
                       pltpu.VMEM((_CWA,), _f32),
                       pltpu.VMEM((_CA, 128), _f32),
                       pltpu.VMEM((16,), _f32),
                       pltpu.VMEM((16, 128), _f32),
                       pltpu.VMEM_SHARED((_RACC, 128), _f32),
                       pltpu.SemaphoreType.DMA],
    )


def _agg1_body(src2, dst2, w_hbm, h0, h1, h2, h3, o0, o1, o2, o3,
               sidx, didx, didx2, wflat, rows, zb, acc_sh, sem):
    c = lax.axis_index("c")
    s = lax.axis_index("s")
    _fill_zb(zb)

    for p in range(4):
        hp = (h0, h1, h2, h3)[p]
        op = (o0, o1, o2, o3)[p]

        @pl.when(c == p // 2)
        def _pass(hp=hp, op=op, p=p):
            for half in range(2):
                base = half * _RH
                _zero_acc(zb, acc_sh, s)
                plsc.subcore_barrier()

                def _batch(t, _):
                    ch0 = s * _CHB_S + t * _BCH
                    pltpu.sync_copy(src2.at[pl.ds(ch0, _BCH), :], sidx)
                    pltpu.sync_copy(dst2.at[pl.ds(ch0, _BCH), :], didx)
                    pltpu.sync_copy(
                        w_hbm.at[pl.ds(ch0 * _CB * 16, _CWB)], wflat)
                    _clamp_idx(didx, didx2, _BCH, base)

                    def _chunk(b, __):
                        pltpu.async_copy(hp.at[sidx.at[b]], rows, sem).wait()

                        def _edge(j, ___):
                            wrow = wflat[pl.ds((b * _CB + j) * 16, 16)]
                            w0 = wrow[2 * p]
                            w1 = wrow[2 * p + 1]
                            for k in range(8):
                                sl = pl.ds(k * 16, 16)
                                ww = w0 if k < 4 else w1
                                rows[j, sl] = rows[j, sl] * ww
                            return 0
                        lax.fori_loop(0, _CB, _edge, 0)
                        pltpu.sync_copy(rows, acc_sh.at[didx2.at[b]], add=True)
                        return 0
                    lax.fori_loop(0, _BCH, _chunk, 0)
                    return 0
                lax.fori_loop(0, _CHB_S // _BCH, _batch, 0)

                plsc.subcore_barrier()
                pltpu.sync_copy(acc_sh.at[pl.ds(s * _WRH, _WRH), :],
                                op.at[pl.ds(base + s * _WRH, _WRH), :])
                plsc.subcore_barrier()


@functools.cache
def _agg1_kernel():
    return pl.kernel(
        _agg1_body,
        mesh=_sc_mesh(),
        out_type=[jax.ShapeDtypeStruct((_NPAD, 128), _f32)] * 4,
        scratch_types=[pltpu.VMEM((_BCH, _CB), jnp.int32),
                       pltpu.VMEM((_BCH, _CB), jnp.int32),
                       pltpu.VMEM((_BCH, _CB), jnp.int32),
                       pltpu.VMEM((_CWB,), _f32),
                       pltpu.VMEM((_CB, 128), _f32),
                       pltpu.VMEM((16, 128), _f32),
                       pltpu.VMEM_SHARED((_RACC, 128), _f32),
                       pltpu.SemaphoreType.DMA],
    )


def _agg2_body(src2, dst2, w_hbm, h2p, o_out,
               sidx, didx, didx2, wflat, rows, zb, acc_sh, sem):
    c = lax.axis_index("c")
    s = lax.axis_index("s")
    wid = s * _NC + c
    _fill_zb(zb)

    for half in range(2):
        base = half * _RH
        _zero_acc(zb, acc_sh, s)
        plsc.subcore_barrier()

        def _batch(t, _):
            ch0 = wid * _CHB_W + t * _BCH
            pltpu.sync_copy(src2.at[pl.ds(ch0, _BCH), :], sidx)
            pltpu.sync_copy(dst2.at[pl.ds(ch0, _BCH), :], didx)
            pltpu.sync_copy(w_hbm.at[pl.ds(ch0 * _CB * 16, _CWB)], wflat)
            _clamp_idx(didx, didx2, _BCH, base)

            def _chunk(b, __):
                pltpu.async_copy(h2p.at[sidx.at[b]], rows, sem).wait()

                def _edge(j, ___):
                    wrow = wflat[pl.ds((b * _CB + j) * 16, 16)]
                    w0 = wrow[0]
                    for k in range(8):
                        sl = pl.ds(k * 16, 16)
                        rows[j, sl] = rows[j, sl] * w0
                    return 0
                lax.fori_loop(0, _CB, _edge, 0)
                pltpu.sync_copy(rows, acc_sh.at[didx2.at[b]], add=True)
                return 0
            lax.fori_loop(0, _BCH, _chunk, 0)
            return 0
        lax.fori_loop(0, _CHB_W // _BCH, _batch, 0)

        plsc.subcore_barrier()
        pltpu.sync_copy(acc_sh.at[pl.ds(s * _WRH, _WRH), :],
                        o_out.at[c, pl.ds(base + s * _WRH, _WRH), :])
        plsc.subcore_barrier()


@functools.cache
def _agg2_kernel():
    return pl.kernel(
        _agg2_body,
        mesh=_sc_mesh(),
        out_type=[jax.ShapeDtypeStruct((_NC, _NPAD, 128), _f32)],
        scratch_types=[pltpu.VMEM((_BCH, _CB), jnp.int32),
                       pltpu.VMEM((_BCH, _CB), jnp.int32),
                       pltpu.VMEM((_BCH, _CB), jnp.int32),
                       pltpu.VMEM((_CWB,), _f32),
                       pltpu.VMEM((_CB, 128), _f32),
                       pltpu.VMEM((16, 128), _f32),
                       pltpu.VMEM_SHARED((_RACC, 128), _f32),
                       pltpu.SemaphoreType.DMA],
    )


_BLK = 1000


def _l1_body(x_ref, w_ref, asrc_ref, adst_ref, h_ref, as_ref, ad_ref):
    h = jnp.dot(x_ref[...], w_ref[...], preferred_element_type=_f32)
    h_ref[...] = h
    hh = h.reshape(h.shape[0], _HEADS, _HID)
    as_ref[...] = (hh * asrc_ref[...][None]).sum(-1)
    ad_ref[...] = (hh * adst_ref[...][None]).sum(-1)


def _layer1_dense(x, W1, a_src1, a_dst1):
    return pl.pallas_call(
        _l1_body,
        grid=(_N // _BLK,),
        in_specs=[
            pl.BlockSpec((_BLK, _DIN), lambda i: (i, 0)),
            pl.BlockSpec((_DIN, _HEADS * _HID), lambda i: (0, 0)),
            pl.BlockSpec((_HEADS, _HID), lambda i: (0, 0)),
            pl.BlockSpec((_HEADS, _HID), lambda i: (0, 0)),
        ],
        out_specs=[
            pl.BlockSpec((_BLK, _HEADS * _HID), lambda i: (i, 0)),
            pl.BlockSpec((_BLK, _HEADS), lambda i: (i, 0)),
            pl.BlockSpec((_BLK, _HEADS), lambda i: (i, 0)),
        ],
        out_shape=[
            jax.ShapeDtypeStruct((_N, _HEADS * _HID), _f32),
            jax.ShapeDtypeStruct((_N, _HEADS), _f32),
            jax.ShapeDtypeStruct((_N, _HEADS), _f32),
        ],
    )(x, W1, a_src1, a_dst1)


def _d2_body(h_ref, den_ref, b_ref, w_ref, a_ref, h2_ref, al_ref):
    den = den_ref[...][:, :, None]
    hn = h_ref[...].reshape(-1, _HEADS, _HID) / (den + 1e-16)
    hn = hn.reshape(h_ref.shape[0], _HEADS * _HID) + b_ref[...]
    hn = jnp.where(hn > 0.0, hn, jnp.exp(hn) - 1.0)
    h2 = jnp.dot(hn, w_ref[...], preferred_element_type=_f32)
    h2_ref[...] = h2
    al_ref[...] = jnp.dot(h2, a_ref[...], preferred_element_type=_f32)


def _dense2(h1s, den1, b1_2d, W2p, A2):
    return pl.pallas_call(
        _d2_body,
        grid=(_N // _BLK,),
        in_specs=[
            pl.BlockSpec((_BLK, _HEADS * _HID), lambda i: (i, 0)),
            pl.BlockSpec((_BLK, _HEADS), lambda i: (i, 0)),
            pl.BlockSpec((1, _HEADS * _HID), lambda i: (0, 0)),
            pl.BlockSpec((_HEADS * _HID, 128), lambda i: (0, 0)),
            pl.BlockSpec((128, 128), lambda i: (0, 0)),
        ],
        out_specs=[
            pl.BlockSpec((_BLK, 128), lambda i: (i, 0)),
            pl.BlockSpec((_BLK, 128), lambda i: (i, 0)),
        ],
        out_shape=[
            jax.ShapeDtypeStruct((_N, 128), _f32),
            jax.ShapeDtypeStruct((_N, 128), _f32),
        ],
    )(h1s, den1, b1_2d, W2p, A2)


def _epi_body(o_ref, d_ref, b_ref, out_ref):
    d = d_ref[...][:, 0:1]
    v = o_ref[...] / (d + 1e-16) + b_ref[...]
    v = jnp.where(v > 0.0, v, jnp.exp(v) - 1.0)
    mask = lax.broadcasted_iota(jnp.int32, v.shape, 1) < _NCLS
    vm = jnp.where(mask, v, -jnp.inf)
    m = jnp.max(vm, axis=1, keepdims=True)
    sm = jnp.sum(jnp.where(mask, jnp.exp(vm - m), 0.0), axis=1, keepdims=True)
    out_ref[...] = v - (jnp.log(sm) + m)


def _epilogue(out2, den16, b2p):
    return pl.pallas_call(
        _epi_body,
        grid=(_N // _BLK,),
        in_specs=[
            pl.BlockSpec((_BLK, 128), lambda i: (i, 0)),
            pl.BlockSpec((_BLK, _HEADS), lambda i: (i, 0)),
            pl.BlockSpec((1, 128), lambda i: (0, 0)),
        ],
        out_specs=pl.BlockSpec((_BLK, 128), lambda i: (i, 0)),
        out_shape=jax.ShapeDtypeStruct((_N, 128), _f32),
    )(out2, den16, b2p)


def kernel(x, edge_index, W1, a_src1, a_dst1, b1, W2, a_src2, a_dst2, b2):
    src = edge_index[0]
    dst = edge_index[1]

    h, als, ald = _layer1_dense(x, W1, a_src1, a_dst1)
    g8 = jnp.maximum(jnp.max(als, axis=0) + jnp.max(ald, axis=0), 0.0)
    g16 = jnp.pad(g8, (0, 8))

    src_a = jnp.concatenate(
        [src, jnp.zeros((_EPAD - _E,), jnp.int32)]).reshape(_CHA_TOT, _CA)
    dst_a = jnp.concatenate(
        [dst, jnp.full((_EPAD - _E,), _N, jnp.int32)]).reshape(_CHA_TOT, _CA)
    src_b = src_a.reshape(_CHB_TOT, _CB)
    dst_b = dst_a.reshape(_CHB_TOT, _CB)

    als_p = jnp.pad(als, ((0, _NPAD - _N), (0, 128 - _HEADS)))
    ald_p = jnp.pad(ald, ((0, _NPAD - _N), (0, 128 - _HEADS)))
    w1, den1p = _attn_kernel()(src_a, dst_a, als_p, ald_p, g16)
    den1 = (den1p[0] + den1p[1])[:_N, :_HEADS]

    hp = [jnp.pad(h[:, 128 * i:128 * (i + 1)], ((0, _NPAD - _N), (0, 0)))
          for i in range(4)]
    o0, o1, o2, o3 = _agg1_kernel()(src_b, dst_b, w1,
                                    hp[0], hp[1], hp[2], hp[3])
    h1s = jnp.concatenate([o0[:_N], o1[:_N], o2[:_N], o3[:_N]], axis=1)

    b1_2d = b1.reshape(1, _HEADS * _HID)
    W2p = jnp.pad(W2, ((0, 0), (0, 128 - _NCLS)))
    A2 = jnp.zeros((128, 128), _f32)
    A2 = A2.at[:_NCLS, 0].set(a_src2[0])
    A2 = A2.at[:_NCLS, 1].set(a_dst2[0])
    h2, al2 = _dense2(h1s, den1, b1_2d, W2p, A2)

    as2 = al2[:, 0]
    ad2 = al2[:, 1]
    g2 = jnp.maximum(jnp.max(as2) + jnp.max(ad2), 0.0)
    g16b = jnp.full((16,), g2, _f32)
    als2_p = jnp.zeros((_NPAD, 128), _f32).at[:_N, 0].set(as2)
    ald2_p = jnp.zeros((_NPAD, 128), _f32).at[:_N, 0].set(ad2)
    w2, den2p = _attn_kernel()(src_a, dst_a, als2_p, ald2_p, g16b)
    den2 = (den2p[0] + den2p[1])[:_N, 0]

    h2p = jnp.pad(h2, ((0, _NPAD - _N), (0, 0)))
    o2p = _agg2_kernel()(src_b, dst_b, w2, h2p)
    if isinstance(o2p, (list, tuple)):
        o2p = o2p[0]
    out2 = (o2p[0] + o2p[1])[:_N]

    den16 = jnp.pad(den2[:, None], ((0, 0), (0, _HEADS - 1)))
    b2p = jnp.pad(b2, (0, 128 - _NCLS)).reshape(1, 128)
    out = _epilogue(out2, den16, b2p)
    return out[:, :_NCLS]


# speedup vs baseline: 6.5558x; 1.3133x over previous
import functools

import jax
import jax.numpy as jnp
from jax import lax
from jax.experimental import pallas as pl
from jax.experimental.pallas import tpu as pltpu
from jax.experimental.pallas import tpu_sc as plsc

_N = 10000
_E = 320000
_DIN = 128
_HID = 64
_HEADS = 8
_NCLS = 40

_NC = 2
_NS = 16
_NPAD = 10240
_EPAD = 327680
_RH = 5120
_RACC = 5376
_ROOB = 5200
_BCH = 8

_CA = 64
_CHA_TOT = _EPAD // _CA
_CHA_W = _CHA_TOT // (_NC * _NS)
_CWA = _BCH * _CA * 16

_CB = 128
_CHB_TOT = _EPAD // _CB
_CHB_W = _CHB_TOT // (_NC * _NS)
_CHB_S = _CHB_TOT // _NS
_CWB = _BCH * _CB * 16

_ZRA = _RACC // _NS
_WRH = _RH // _NS

_f32 = jnp.float32


@functools.cache
def _sc_mesh():
    return plsc.VectorSubcoreMesh(core_axis_name="c", subcore_axis_name="s")


def _zero16():
    return jnp.zeros((16,), _f32)


def _fill_zb(zb):
    def _zrow(i, _):
        for k in range(8):
            zb[i, pl.ds(k * 16, 16)] = _zero16()
        return 0
    lax.fori_loop(0, 16, _zrow, 0)


def _zero_acc(zb, acc_sh, s):
    def _zcp(t, _):
        pltpu.sync_copy(zb, acc_sh.at[pl.ds(s * _ZRA + t * 16, 16), :])
        return 0
    lax.fori_loop(0, _ZRA // 16, _zcp, 0)


def _clamp_idx(didx, didx2, nrows, base, groups=8):
    def _row(r, _):
        for k in range(groups):
            sl = pl.ds(k * 16, 16)
            v = didx[r, sl] - base
            oob = (v < 0) | (v >= _RH)
            didx2[r, sl] = jnp.where(oob, _ROOB, v)
        return 0
    lax.fori_loop(0, nrows, _row, 0)


def _attn_body(src2, dst2, als, ald, g16, w_out, den_out,
               sidx, didx, didx2, asv, adv, wflat, wpad, gv, zb, den_sh, sem):
    c = lax.axis_index("c")
    s = lax.axis_index("s")
    wid = s * _NC + c
    pltpu.sync_copy(g16, gv)
    _fill_zb(zb)

    def _zpad(j, _):
        for k in range(8):
            wpad[j, pl.ds(k * 16, 16)] = _zero16()
        return 0
    lax.fori_loop(0, _CA, _zpad, 0)

    g = gv[...]

    for half in range(2):
        base = half * _RH
        _zero_acc(zb, den_sh, s)
        plsc.subcore_barrier()

        def _batch(t, _, half=half):
            ch0 = wid * _CHA_W + t * _BCH
            pltpu.sync_copy(dst2.at[pl.ds(ch0, _BCH), :], didx)
            _clamp_idx(didx, didx2, _BCH, base, groups=_CA // 16)
            if half == 0:
                pltpu.sync_copy(src2.at[pl.ds(ch0, _BCH), :], sidx)
            else:
                pltpu.sync_copy(w_out.at[pl.ds(ch0 * _CA * 16, _CWA)], wflat)

            def _chunk(b, __):
                if half == 0:
                    pltpu.async_copy(als.at[sidx.at[b]], asv, sem).wait()
                    pltpu.async_copy(ald.at[didx.at[b]], adv, sem).wait()

                    def _edge(j, ___):
                        v = asv[j, pl.ds(0, 16)] + adv[j, pl.ds(0, 16)]
                        e = jnp.where(v >= 0.0, v, 0.2 * v)
                        w = jnp.exp(e - g)
                        wflat[pl.ds((b * _CA + j) * 16, 16)] = w
                        wpad[j, pl.ds(0, 16)] = w
                        return 0
                else:
                    def _edge(j, ___):
                        wpad[j, pl.ds(0, 16)] = \
                            wflat[pl.ds((b * _CA + j) * 16, 16)]
                        return 0
                lax.fori_loop(0, _CA, _edge, 0)
                pltpu.sync_copy(wpad, den_sh.at[didx2.at[b]], add=True)
                return 0
            lax.fori_loop(0, _BCH, _chunk, 0)
            if half == 0:
                pltpu.sync_copy(wflat, w_out.at[pl.ds(ch0 * _CA * 16, _CWA)])
            return 0
        lax.fori_loop(0, _CHA_W // _BCH, _batch, 0)

        plsc.subcore_barrier()
        pltpu.sync_copy(den_sh.at[pl.ds(s * _WRH, _WRH), :],
                        den_out.at[c, pl.ds(base + s * _WRH, _WRH), :])
        plsc.subcore_barrier()


@functools.cache
def _attn_kernel():
    return pl.kernel(
        _attn_body,
        mesh=_sc_mesh(),
        out_type=[jax.ShapeDtypeStruct((_EPAD * 16,), _f32),
                  jax.ShapeDtypeStruct((_NC, _NPAD, 128), _f32)],
        scratch_types=[pltpu.VMEM((_BCH, _CA), jnp.int32),
                       pltpu.VMEM((_BCH, _CA), jnp.int32),
                       pltpu.VMEM((_BCH, _CA), jnp.int32),
                       pltpu.VMEM((_CA, 128), _f32),
                       pltpu.VMEM((_CA, 128), _f32),
                       pltpu.VMEM((_CWA,), _f32),
                       pltpu.VMEM((_CA, 128), _f32),
                       pltpu.VMEM((16,), _f32),
                       pltpu.VMEM((16, 128), _f32),
                       pltpu.VMEM_SHARED((_RACC, 128), _f32),
                       pltpu.SemaphoreType.DMA],
    )


def _attn2_body(src2, dst2, als, ald, g16, w_out,
                sidx, didx, asv, adv, wflat, gv, sem):
    c = lax.axis_index("c")
    s = lax.axis_index("s")
    wid = s * _NC + c
    pltpu.sync_copy(g16, gv)
    g = gv[...]

    def _batch(t, _):
        ch0 = wid * _CHA_W + t * _BCH
        pltpu.sync_copy(src2.at[pl.ds(ch0, _BCH), :], sidx)
        pltpu.sync_copy(dst2.at[pl.ds(ch0, _BCH), :], didx)

        def _chunk(b, __):
            pltpu.async_copy(als.at[sidx.at[b]], asv, sem).wait()
            pltpu.async_copy(ald.at[didx.at[b]], adv, sem).wait()

            def _edge(j, ___):
                v = asv[j, pl.ds(0, 16)] + adv[j, pl.ds(0, 16)]
                e = jnp.where(v >= 0.0, v, 0.2 * v)
                wflat[pl.ds((b * _CA + j) * 16, 16)] = jnp.exp(e - g)
                return 0
            lax.fori_loop(0, _CA, _edge, 0)
            return 0
        lax.fori_loop(0, _BCH, _chunk, 0)
        pltpu.sync_copy(wflat, w_out.at[pl.ds(ch0 * _CA * 16, _CWA)])
        return 0
    lax.fori_loop(0, _CHA_W // _BCH, _batch, 0)


@functools.cache
def _attn2_kernel():
    return pl.kernel(
        _attn2_body,
        mesh=_sc_mesh(),
        out_type=[jax.ShapeDtypeStruct((_EPAD * 16,), _f32)],
        scratch_types=[pltpu.VMEM((_BCH, _CA), jnp.int32),
                       pltpu.VMEM((_BCH, _CA), jnp.int32),
                       pltpu.VMEM((_CA, 128), _f32),
                       pltpu.VMEM((_CA, 128), _f32),
                       pltpu.VMEM((_CWA,), _f32),
                       pltpu.VMEM((16,), _f32),
                       pltpu.SemaphoreType.DMA],
    )


def _agg1_body(src2, dst2, w_hbm, h0, h1, h2, h3, o0, o1, o2, o3,
               sidx, didx, didx2, wflat, rows, zb, acc_sh, sem):
    c = lax.axis_index("c")
    s = lax.axis_index("s")
    _fill_zb(zb)

    for p in range(4):
        hp = (h0, h1, h2, h3)[p]
        op = (o0, o1, o2, o3)[p]

        @pl.when(c == p // 2)
        def _pass(hp=hp, op=op, p=p):
            for half in range(2):
                base = half * _RH
                _zero_acc(zb, acc_sh, s)
                plsc.subcore_barrier()

                def _batch(t, _):
                    ch0 = s * _CHB_S + t * _BCH
                    pltpu.sync_copy(src2.at[pl.ds(ch0, _BCH), :], sidx)
                    pltpu.sync_copy(dst2.at[pl.ds(ch0, _BCH), :], didx)
                    pltpu.sync_copy(
                        w_hbm.at[pl.ds(ch0 * _CB * 16, _CWB)], wflat)
                    _clamp_idx(didx, didx2, _BCH, base)

                    def _chunk(b, __):
                        pltpu.async_copy(hp.at[sidx.at[b]], rows, sem).wait()

                        def _edge(j, ___):
                            wrow = wflat[pl.ds((b * _CB + j) * 16, 16)]
                            w0 = wrow[2 * p]
                            w1 = wrow[2 * p + 1]
                            for k in range(8):
                                sl = pl.ds(k * 16, 16)
                                ww = w0 if k < 4 else w1
                                rows[j, sl] = rows[j, sl] * ww
                            return 0
                        lax.fori_loop(0, _CB, _edge, 0)
                        pltpu.sync_copy(rows, acc_sh.at[didx2.at[b]], add=True)
                        return 0
                    lax.fori_loop(0, _BCH, _chunk, 0)
                    return 0
                lax.fori_loop(0, _CHB_S // _BCH, _batch, 0)

                plsc.subcore_barrier()
                pltpu.sync_copy(acc_sh.at[pl.ds(s * _WRH, _WRH), :],
                                op.at[pl.ds(base + s * _WRH, _WRH), :])
                plsc.subcore_barrier()


@functools.cache
def _agg1_kernel():
    return pl.kernel(
        _agg1_body,
        mesh=_sc_mesh(),
        out_type=[jax.ShapeDtypeStruct((_NPAD, 128), _f32)] * 4,
        scratch_types=[pltpu.VMEM((_BCH, _CB), jnp.int32),
                       pltpu.VMEM((_BCH, _CB), jnp.int32),
                       pltpu.VMEM((_BCH, _CB), jnp.int32),
                       pltpu.VMEM((_CWB,), _f32),
                       pltpu.VMEM((_CB, 128), _f32),
                       pltpu.VMEM((16, 128), _f32),
                       pltpu.VMEM_SHARED((_RACC, 128), _f32),
                       pltpu.SemaphoreType.DMA],
    )


def _agg2_body(src2, dst2, w_hbm, h2p, o_out,
               sidx, didx, didx2, wflat, rows, zb, acc_sh, sem):
    c = lax.axis_index("c")
    s = lax.axis_index("s")
    wid = s * _NC + c
    _fill_zb(zb)

    for half in range(2):
        base = half * _RH
        _zero_acc(zb, acc_sh, s)
        plsc.subcore_barrier()

        def _batch(t, _):
            ch0 = wid * _CHB_W + t * _BCH
            pltpu.sync_copy(src2.at[pl.ds(ch0, _BCH), :], sidx)
            pltpu.sync_copy(dst2.at[pl.ds(ch0, _BCH), :], didx)
            pltpu.sync_copy(w_hbm.at[pl.ds(ch0 * _CB * 16, _CWB)], wflat)
            _clamp_idx(didx, didx2, _BCH, base)

            def _chunk(b, __):
                pltpu.async_copy(h2p.at[sidx.at[b]], rows, sem).wait()

                def _edge(j, ___):
                    wrow = wflat[pl.ds((b * _CB + j) * 16, 16)]
                    w0 = wrow[0]
                    for k in range(8):
                        sl = pl.ds(k * 16, 16)
                        rows[j, sl] = rows[j, sl] * w0
                    return 0
                lax.fori_loop(0, _CB, _edge, 0)
                pltpu.sync_copy(rows, acc_sh.at[didx2.at[b]], add=True)
                return 0
            lax.fori_loop(0, _BCH, _chunk, 0)
            return 0
        lax.fori_loop(0, _CHB_W // _BCH, _batch, 0)

        plsc.subcore_barrier()
        pltpu.sync_copy(acc_sh.at[pl.ds(s * _WRH, _WRH), :],
                        o_out.at[c, pl.ds(base + s * _WRH, _WRH), :])
        plsc.subcore_barrier()


@functools.cache
def _agg2_kernel():
    return pl.kernel(
        _agg2_body,
        mesh=_sc_mesh(),
        out_type=[jax.ShapeDtypeStruct((_NC, _NPAD, 128), _f32)],
        scratch_types=[pltpu.VMEM((_BCH, _CB), jnp.int32),
                       pltpu.VMEM((_BCH, _CB), jnp.int32),
                       pltpu.VMEM((_BCH, _CB), jnp.int32),
                       pltpu.VMEM((_CWB,), _f32),
                       pltpu.VMEM((_CB, 128), _f32),
                       pltpu.VMEM((16, 128), _f32),
                       pltpu.VMEM_SHARED((_RACC, 128), _f32),
                       pltpu.SemaphoreType.DMA],
    )


_BLK = 1000


def _l1_body(x_ref, w_ref, asrc_ref, adst_ref, h_ref, as_ref, ad_ref):
    h = jnp.dot(x_ref[...], w_ref[...], preferred_element_type=_f32)
    h_ref[...] = h
    hh = h.reshape(h.shape[0], _HEADS, _HID)
    as_ref[...] = (hh * asrc_ref[...][None]).sum(-1)
    ad_ref[...] = (hh * adst_ref[...][None]).sum(-1)


def _layer1_dense(x, W1, a_src1, a_dst1):
    return pl.pallas_call(
        _l1_body,
        grid=(_N // _BLK,),
        in_specs=[
            pl.BlockSpec((_BLK, _DIN), lambda i: (i, 0)),
            pl.BlockSpec((_DIN, _HEADS * _HID), lambda i: (0, 0)),
            pl.BlockSpec((_HEADS, _HID), lambda i: (0, 0)),
            pl.BlockSpec((_HEADS, _HID), lambda i: (0, 0)),
        ],
        out_specs=[
            pl.BlockSpec((_BLK, _HEADS * _HID), lambda i: (i, 0)),
            pl.BlockSpec((_BLK, _HEADS), lambda i: (i, 0)),
            pl.BlockSpec((_BLK, _HEADS), lambda i: (i, 0)),
        ],
        out_shape=[
            jax.ShapeDtypeStruct((_N, _HEADS * _HID), _f32),
            jax.ShapeDtypeStruct((_N, _HEADS), _f32),
            jax.ShapeDtypeStruct((_N, _HEADS), _f32),
        ],
    )(x, W1, a_src1, a_dst1)


def _d2_body(h_ref, den_ref, b_ref, w_ref, a_ref, h2_ref, al_ref):
    den = den_ref[...][:, :, None]
    hn = h_ref[...].reshape(-1, _HEADS, _HID) / (den + 1e-16)
    hn = hn.reshape(h_ref.shape[0], _HEADS * _HID) + b_ref[...]
    hn = jnp.where(hn > 0.0, hn, jnp.exp(hn) - 1.0)
    h2 = jnp.dot(hn, w_ref[...], preferred_element_type=_f32)
    h2_ref[...] = h2
    al_ref[...] = jnp.dot(h2, a_ref[...], preferred_element_type=_f32)


def _dense2(h1s, den1, b1_2d, W2p, A2):
    return pl.pallas_call(
        _d2_body,
        grid=(_N // _BLK,),
        in_specs=[
            pl.BlockSpec((_BLK, _HEADS * _HID), lambda i: (i, 0)),
            pl.BlockSpec((_BLK, _HEADS), lambda i: (i, 0)),
            pl.BlockSpec((1, _HEADS * _HID), lambda i: (0, 0)),
            pl.BlockSpec((_HEADS * _HID, 128), lambda i: (0, 0)),
            pl.BlockSpec((128, 128), lambda i: (0, 0)),
        ],
        out_specs=[
            pl.BlockSpec((_BLK, 128), lambda i: (i, 0)),
            pl.BlockSpec((_BLK, 128), lambda i: (i, 0)),
        ],
        out_shape=[
            jax.ShapeDtypeStruct((_N, 128), _f32),
            jax.ShapeDtypeStruct((_N, 128), _f32),
        ],
    )(h1s, den1, b1_2d, W2p, A2)


def _epi_body(o_ref, d_ref, b_ref, out_ref):
    d = d_ref[...][:, 0:1]
    v = o_ref[...] / (d + 1e-16) + b_ref[...]
    v = jnp.where(v > 0.0, v, jnp.exp(v) - 1.0)
    mask = lax.broadcasted_iota(jnp.int32, v.shape, 1) < _NCLS
    vm = jnp.where(mask, v, -jnp.inf)
    m = jnp.max(vm, axis=1, keepdims=True)
    sm = jnp.sum(jnp.where(mask, jnp.exp(vm - m), 0.0), axis=1, keepdims=True)
    out_ref[...] = v - (jnp.log(sm) + m)


def _epilogue(out2, den16, b2p):
    return pl.pallas_call(
        _epi_body,
        grid=(_N // _BLK,),
        in_specs=[
            pl.BlockSpec((_BLK, 128), lambda i: (i, 0)),
            pl.BlockSpec((_BLK, _HEADS), lambda i: (i, 0)),
            pl.BlockSpec((1, 128), lambda i: (0, 0)),
        ],
        out_specs=pl.BlockSpec((_BLK, 128), lambda i: (i, 0)),
        out_shape=jax.ShapeDtypeStruct((_N, 128), _f32),
    )(out2, den16, b2p)


def kernel(x, edge_index, W1, a_src1, a_dst1, b1, W2, a_src2, a_dst2, b2):
    src = edge_index[0]
    dst = edge_index[1]

    h, als, ald = _layer1_dense(x, W1, a_src1, a_dst1)
    g8 = jnp.maximum(jnp.max(als, axis=0) + jnp.max(ald, axis=0), 0.0)
    g16 = jnp.pad(g8, (0, 8))

    src_a = jnp.concatenate(
        [src, jnp.zeros((_EPAD - _E,), jnp.int32)]).reshape(_CHA_TOT, _CA)
    dst_a = jnp.concatenate(
        [dst, jnp.full((_EPAD - _E,), _N, jnp.int32)]).reshape(_CHA_TOT, _CA)
    src_b = src_a.reshape(_CHB_TOT, _CB)
    dst_b = dst_a.reshape(_CHB_TOT, _CB)

    als_p = jnp.pad(als, ((0, _NPAD - _N), (0, 128 - _HEADS)))
    ald_p = jnp.pad(ald, ((0, _NPAD - _N), (0, 128 - _HEADS)))
    w1, den1p = _attn_kernel()(src_a, dst_a, als_p, ald_p, g16)
    den1 = (den1p[0] + den1p[1])[:_N, :_HEADS]

    hp = [jnp.pad(h[:, 128 * i:128 * (i + 1)], ((0, _NPAD - _N), (0, 0)))
          for i in range(4)]
    o0, o1, o2, o3 = _agg1_kernel()(src_b, dst_b, w1,
                                    hp[0], hp[1], hp[2], hp[3])
    h1s = jnp.concatenate([o0[:_N], o1[:_N], o2[:_N], o3[:_N]], axis=1)

    b1_2d = b1.reshape(1, _HEADS * _HID)
    W2p = jnp.pad(W2, ((0, 0), (0, 128 - _NCLS)))
    A2 = jnp.zeros((128, 128), _f32)
    A2 = A2.at[:_NCLS, 0].set(a_src2[0])
    A2 = A2.at[:_NCLS, 1].set(a_dst2[0])
    h2, al2 = _dense2(h1s, den1, b1_2d, W2p, A2)

    as2 = al2[:, 0]
    ad2 = al2[:, 1]
    g2 = jnp.maximum(jnp.max(as2) + jnp.max(ad2), 0.0)
    g16b = jnp.full((16,), g2, _f32)
    als2_p = jnp.zeros((_NPAD, 128), _f32).at[:_N, 0].set(as2)
    ald2_p = jnp.zeros((_NPAD, 128), _f32).at[:_N, 0].set(ad2)
    w2 = _attn2_kernel()(src_a, dst_a, als2_p, ald2_p, g16b)
    if isinstance(w2, (list, tuple)):
        w2 = w2[0]

    h2p = jnp.pad(h2, ((0, _NPAD - _N), (0, 0))).at[:_N, _NCLS].set(1.0)
    o2p = _agg2_kernel()(src_b, dst_b, w2, h2p)
    if isinstance(o2p, (list, tuple)):
        o2p = o2p[0]
    out2 = (o2p[0] + o2p[1])[:_N]
    den2 = out2[:, _NCLS]

    den16 = jnp.pad(den2[:, None], ((0, 0), (0, _HEADS - 1)))
    b2p = jnp.pad(b2, (0, 128 - _NCLS)).reshape(1, 128)
    out = _epilogue(out2, den16, b2p)
    return out[:, :_NCLS]


# speedup vs baseline: 7.6173x; 1.1619x over previous
import functools

import jax
import jax.numpy as jnp
from jax import lax
from jax.experimental import pallas as pl
from jax.experimental.pallas import tpu as pltpu
from jax.experimental.pallas import tpu_sc as plsc

_N = 10000
_E = 320000
_DIN = 128
_HID = 64
_HEADS = 8
_NCLS = 40

_NC = 2
_NS = 16
_NPAD = 10240
_EPAD = 327680
_RH = 5120
_RACC = 5376
_ROOB = 5200
_BCH = 8

_CA = 64
_CHA_TOT = _EPAD // _CA
_CHA_W = _CHA_TOT // (_NC * _NS)
_CWA = _BCH * _CA * 16

_CB = 128
_CHB_TOT = _EPAD // _CB
_CHB_W = _CHB_TOT // (_NC * _NS)
_CHB_S = _CHB_TOT // _NS
_CWB = _BCH * _CB * 16

_ZRA = _RACC // _NS
_WRH = _RH // _NS

_f32 = jnp.float32


@functools.cache
def _sc_mesh():
    return plsc.VectorSubcoreMesh(core_axis_name="c", subcore_axis_name="s")


def _zero16():
    return jnp.zeros((16,), _f32)


def _fill_zb(zb):
    def _zrow(i, _):
        for k in range(8):
            zb[i, pl.ds(k * 16, 16)] = _zero16()
        return 0
    lax.fori_loop(0, 16, _zrow, 0)


def _zero_acc(zb, acc_sh, s):
    def _zcp(t, _):
        pltpu.sync_copy(zb, acc_sh.at[pl.ds(s * _ZRA + t * 16, 16), :])
        return 0
    lax.fori_loop(0, _ZRA // 16, _zcp, 0)


def _clamp_idx(didx, didx2, nrows, base, groups=8):
    def _row(r, _):
        for k in range(groups):
            sl = pl.ds(k * 16, 16)
            v = didx[r, sl] - base
            oob = (v < 0) | (v >= _RH)
            didx2[r, sl] = jnp.where(oob, _ROOB, v)
        return 0
    lax.fori_loop(0, nrows, _row, 0)


def _attn_body(src2, dst2, als, ald, g16, w_out, den_out,
               sidx, didx, didx2, asv, adv, wflat, wpad, gv, zb, den_sh, sem):
    c = lax.axis_index("c")
    s = lax.axis_index("s")
    wid = s * _NC + c
    pltpu.sync_copy(g16, gv)
    _fill_zb(zb)

    def _zpad(j, _):
        for k in range(8):
            wpad[j, pl.ds(k * 16, 16)] = _zero16()
        return 0
    lax.fori_loop(0, _CA, _zpad, 0)

    g = gv[...]

    for half in range(2):
        base = half * _RH
        _zero_acc(zb, den_sh, s)
        plsc.subcore_barrier()

        def _batch(t, _, half=half):
            ch0 = wid * _CHA_W + t * _BCH
            pltpu.sync_copy(dst2.at[pl.ds(ch0, _BCH), :], didx)
            _clamp_idx(didx, didx2, _BCH, base, groups=_CA // 16)
            if half == 0:
                pltpu.sync_copy(src2.at[pl.ds(ch0, _BCH), :], sidx)
            else:
                pltpu.sync_copy(w_out.at[pl.ds(ch0 * _CA * 16, _CWA)], wflat)

            def _chunk(b, __):
                if half == 0:
                    pltpu.async_copy(als.at[sidx.at[b]], asv, sem).wait()
                    pltpu.async_copy(ald.at[didx.at[b]], adv, sem).wait()

                    def _edge(j, ___):
                        v = asv[j, pl.ds(0, 16)] + adv[j, pl.ds(0, 16)]
                        e = jnp.where(v >= 0.0, v, 0.2 * v)
                        w = jnp.exp(e - g)
                        wflat[pl.ds((b * _CA + j) * 16, 16)] = w
                        wpad[j, pl.ds(0, 16)] = w
                        return 0
                else:
                    def _edge(j, ___):
                        wpad[j, pl.ds(0, 16)] = \
                            wflat[pl.ds((b * _CA + j) * 16, 16)]
                        return 0
                lax.fori_loop(0, _CA, _edge, 0)
                pltpu.sync_copy(wpad, den_sh.at[didx2.at[b]], add=True)
                return 0
            lax.fori_loop(0, _BCH, _chunk, 0)
            if half == 0:
                pltpu.sync_copy(wflat, w_out.at[pl.ds(ch0 * _CA * 16, _CWA)])
            return 0
        lax.fori_loop(0, _CHA_W // _BCH, _batch, 0)

        plsc.subcore_barrier()
        pltpu.sync_copy(den_sh.at[pl.ds(s * _WRH, _WRH), :],
                        den_out.at[c, pl.ds(base + s * _WRH, _WRH), :])
        plsc.subcore_barrier()


@functools.cache
def _attn_kernel():
    return pl.kernel(
        _attn_body,
        mesh=_sc_mesh(),
        out_type=[jax.ShapeDtypeStruct((_EPAD * 16,), _f32),
                  jax.ShapeDtypeStruct((_NC, _NPAD, 128), _f32)],
        scratch_types=[pltpu.VMEM((_BCH, _CA), jnp.int32),
                       pltpu.VMEM((_BCH, _CA), jnp.int32),
                       pltpu.VMEM((_BCH, _CA), jnp.int32),
                       pltpu.VMEM((_CA, 128), _f32),
                       pltpu.VMEM((_CA, 128), _f32),
                       pltpu.VMEM((_CWA,), _f32),
                       pltpu.VMEM((_CA, 128), _f32),
                       pltpu.VMEM((16,), _f32),
                       pltpu.VMEM((16, 128), _f32),
                       pltpu.VMEM_SHARED((_RACC, 128), _f32),
                       pltpu.SemaphoreType.DMA],
    )


def _attn2_body(src2, dst2, als, ald, g16, w_out,
                sidx, didx, asv, adv, wflat, gv, sem):
    c = lax.axis_index("c")
    s = lax.axis_index("s")
    wid = s * _NC + c
    pltpu.sync_copy(g16, gv)
    g = gv[...]

    def _batch(t, _):
        ch0 = wid * _CHA_W + t * _BCH
        pltpu.sync_copy(src2.at[pl.ds(ch0, _BCH), :], sidx)
        pltpu.sync_copy(dst2.at[pl.ds(ch0, _BCH), :], didx)

        def _chunk(b, __):
            pltpu.async_copy(als.at[sidx.at[b]], asv, sem).wait()
            pltpu.async_copy(ald.at[didx.at[b]], adv, sem).wait()

            def _edge(j, ___):
                v = asv[j, pl.ds(0, 16)] + adv[j, pl.ds(0, 16)]
                e = jnp.where(v >= 0.0, v, 0.2 * v)
                wflat[pl.ds((b * _CA + j) * 16, 16)] = jnp.exp(e - g)
                return 0
            lax.fori_loop(0, _CA, _edge, 0)
            return 0
        lax.fori_loop(0, _BCH, _chunk, 0)
        pltpu.sync_copy(wflat, w_out.at[pl.ds(ch0 * _CA * 16, _CWA)])
        return 0
    lax.fori_loop(0, _CHA_W // _BCH, _batch, 0)


@functools.cache
def _attn2_kernel():
    return pl.kernel(
        _attn2_body,
        mesh=_sc_mesh(),
        out_type=[jax.ShapeDtypeStruct((_EPAD * 16,), _f32)],
        scratch_types=[pltpu.VMEM((_BCH, _CA), jnp.int32),
                       pltpu.VMEM((_BCH, _CA), jnp.int32),
                       pltpu.VMEM((_CA, 128), _f32),
                       pltpu.VMEM((_CA, 128), _f32),
                       pltpu.VMEM((_CWA,), _f32),
                       pltpu.VMEM((16,), _f32),
                       pltpu.SemaphoreType.DMA],
    )


def _scaled_scatter(hp, acc_sh, sidx, didx2, rows_a, rows_b, sem, scale_fn):
    bufs = (rows_a, rows_b)
    handles = [None] * _BCH
    handles[0] = pltpu.async_copy(hp.at[sidx.at[0]], bufs[0], sem)
    for b in range(_BCH):
        handles[b].wait()
        if b + 1 < _BCH:
            handles[b + 1] = pltpu.async_copy(
                hp.at[sidx.at[b + 1]], bufs[(b + 1) % 2], sem)
        rbuf = bufs[b % 2]

        def _edge(j, ___, b=b, rbuf=rbuf):
            scale_fn(rbuf, b, j)
            return 0
        lax.fori_loop(0, _CB, _edge, 0)
        pltpu.sync_copy(rbuf, acc_sh.at[didx2.at[b]], add=True)


def _agg1_body(src2, dst2, w_hbm, h0, h1, h2, h3, o0, o1, o2, o3,
               sidx, didx, didx2, wflat, rows_a, rows_b, zb, acc_sh, sem):
    c = lax.axis_index("c")
    s = lax.axis_index("s")
    _fill_zb(zb)

    for p in range(4):
        hp = (h0, h1, h2, h3)[p]
        op = (o0, o1, o2, o3)[p]

        @pl.when(c == p // 2)
        def _pass(hp=hp, op=op, p=p):
            def _scale(rbuf, b, j, p=p):
                wrow = wflat[pl.ds((b * _CB + j) * 16, 16)]
                w0 = wrow[2 * p]
                w1 = wrow[2 * p + 1]
                for k in range(8):
                    sl = pl.ds(k * 16, 16)
                    ww = w0 if k < 4 else w1
                    rbuf[j, sl] = rbuf[j, sl] * ww

            for half in range(2):
                base = half * _RH
                _zero_acc(zb, acc_sh, s)
                plsc.subcore_barrier()

                def _batch(t, _):
                    ch0 = s * _CHB_S + t * _BCH
                    pltpu.sync_copy(src2.at[pl.ds(ch0, _BCH), :], sidx)
                    pltpu.sync_copy(dst2.at[pl.ds(ch0, _BCH), :], didx)
                    pltpu.sync_copy(
                        w_hbm.at[pl.ds(ch0 * _CB * 16, _CWB)], wflat)
                    _clamp_idx(didx, didx2, _BCH, base)
                    _scaled_scatter(hp, acc_sh, sidx, didx2,
                                    rows_a, rows_b, sem, _scale)
                    return 0
                lax.fori_loop(0, _CHB_S // _BCH, _batch, 0)

                plsc.subcore_barrier()
                pltpu.sync_copy(acc_sh.at[pl.ds(s * _WRH, _WRH), :],
                                op.at[pl.ds(base + s * _WRH, _WRH), :])
                plsc.subcore_barrier()


@functools.cache
def _agg1_kernel():
    return pl.kernel(
        _agg1_body,
        mesh=_sc_mesh(),
        out_type=[jax.ShapeDtypeStruct((_NPAD, 128), _f32)] * 4,
        scratch_types=[pltpu.VMEM((_BCH, _CB), jnp.int32),
                       pltpu.VMEM((_BCH, _CB), jnp.int32),
                       pltpu.VMEM((_BCH, _CB), jnp.int32),
                       pltpu.VMEM((_CWB,), _f32),
                       pltpu.VMEM((_CB, 128), _f32),
                       pltpu.VMEM((_CB, 128), _f32),
                       pltpu.VMEM((16, 128), _f32),
                       pltpu.VMEM_SHARED((_RACC, 128), _f32),
                       pltpu.SemaphoreType.DMA],
    )


def _agg2_body(src2, dst2, w_hbm, h2p, o_out,
               sidx, didx, didx2, wflat, rows_a, rows_b, zb, acc_sh, sem):
    c = lax.axis_index("c")
    s = lax.axis_index("s")
    wid = s * _NC + c
    _fill_zb(zb)

    def _scale(rbuf, b, j):
        wrow = wflat[pl.ds((b * _CB + j) * 16, 16)]
        w0 = wrow[0]
        for k in range(8):
            sl = pl.ds(k * 16, 16)
            rbuf[j, sl] = rbuf[j, sl] * w0

    for half in range(2):
        base = half * _RH
        _zero_acc(zb, acc_sh, s)
        plsc.subcore_barrier()

        def _batch(t, _):
            ch0 = wid * _CHB_W + t * _BCH
            pltpu.sync_copy(src2.at[pl.ds(ch0, _BCH), :], sidx)
            pltpu.sync_copy(dst2.at[pl.ds(ch0, _BCH), :], didx)
            pltpu.sync_copy(w_hbm.at[pl.ds(ch0 * _CB * 16, _CWB)], wflat)
            _clamp_idx(didx, didx2, _BCH, base)
            _scaled_scatter(h2p, acc_sh, sidx, didx2,
                            rows_a, rows_b, sem, _scale)
            return 0
        lax.fori_loop(0, _CHB_W // _BCH, _batch, 0)

        plsc.subcore_barrier()
        pltpu.sync_copy(acc_sh.at[pl.ds(s * _WRH, _WRH), :],
                        o_out.at[c, pl.ds(base + s * _WRH, _WRH), :])
        plsc.subcore_barrier()


@functools.cache
def _agg2_kernel():
    return pl.kernel(
        _agg2_body,
        mesh=_sc_mesh(),
        out_type=[jax.ShapeDtypeStruct((_NC, _NPAD, 128), _f32)],
        scratch_types=[pltpu.VMEM((_BCH, _CB), jnp.int32),
                       pltpu.VMEM((_BCH, _CB), jnp.int32),
                       pltpu.VMEM((_BCH, _CB), jnp.int32),
                       pltpu.VMEM((_CWB,), _f32),
                       pltpu.VMEM((_CB, 128), _f32),
                       pltpu.VMEM((_CB, 128), _f32),
                       pltpu.VMEM((16, 128), _f32),
                       pltpu.VMEM_SHARED((_RACC, 128), _f32),
                       pltpu.SemaphoreType.DMA],
    )


_BLK = 1000


def _l1_body(x_ref, w_ref, asrc_ref, adst_ref, h_ref, as_ref, ad_ref):
    h = jnp.dot(x_ref[...], w_ref[...], preferred_element_type=_f32)
    h_ref[...] = h
    hh = h.reshape(h.shape[0], _HEADS, _HID)
    as_ref[...] = (hh * asrc_ref[...][None]).sum(-1)
    ad_ref[...] = (hh * adst_ref[...][None]).sum(-1)


def _layer1_dense(x, W1, a_src1, a_dst1):
    return pl.pallas_call(
        _l1_body,
        grid=(_N // _BLK,),
        in_specs=[
            pl.BlockSpec((_BLK, _DIN), lambda i: (i, 0)),
            pl.BlockSpec((_DIN, _HEADS * _HID), lambda i: (0, 0)),
            pl.BlockSpec((_HEADS, _HID), lambda i: (0, 0)),
            pl.BlockSpec((_HEADS, _HID), lambda i: (0, 0)),
        ],
        out_specs=[
            pl.BlockSpec((_BLK, _HEADS * _HID), lambda i: (i, 0)),
            pl.BlockSpec((_BLK, _HEADS), lambda i: (i, 0)),
            pl.BlockSpec((_BLK, _HEADS), lambda i: (i, 0)),
        ],
        out_shape=[
            jax.ShapeDtypeStruct((_N, _HEADS * _HID), _f32),
            jax.ShapeDtypeStruct((_N, _HEADS), _f32),
            jax.ShapeDtypeStruct((_N, _HEADS), _f32),
        ],
    )(x, W1, a_src1, a_dst1)


def _d2_body(h_ref, den_ref, b_ref, w_ref, a_ref, h2_ref, al_ref):
    den = den_ref[...][:, :, None]
    hn = h_ref[...].reshape(-1, _HEADS, _HID) / (den + 1e-16)
    hn = hn.reshape(h_ref.shape[0], _HEADS * _HID) + b_ref[...]
    hn = jnp.where(hn > 0.0, hn, jnp.exp(hn) - 1.0)
    h2 = jnp.dot(hn, w_ref[...], preferred_element_type=_f32)
    h2_ref[...] = h2
    al_ref[...] = jnp.dot(h2, a_ref[...], preferred_element_type=_f32)


def _dense2(h1s, den1, b1_2d, W2p, A2):
    return pl.pallas_call(
        _d2_body,
        grid=(_N // _BLK,),
        in_specs=[
            pl.BlockSpec((_BLK, _HEADS * _HID), lambda i: (i, 0)),
            pl.BlockSpec((_BLK, _HEADS), lambda i: (i, 0)),
            pl.BlockSpec((1, _HEADS * _HID), lambda i: (0, 0)),
            pl.BlockSpec((_HEADS * _HID, 128), lambda i: (0, 0)),
            pl.BlockSpec((128, 128), lambda i: (0, 0)),
        ],
        out_specs=[
            pl.BlockSpec((_BLK, 128), lambda i: (i, 0)),
            pl.BlockSpec((_BLK, 128), lambda i: (i, 0)),
        ],
        out_shape=[
            jax.ShapeDtypeStruct((_N, 128), _f32),
            jax.ShapeDtypeStruct((_N, 128), _f32),
        ],
    )(h1s, den1, b1_2d, W2p, A2)


def _epi_body(o_ref, d_ref, b_ref, out_ref):
    d = d_ref[...][:, 0:1]
    v = o_ref[...] / (d + 1e-16) + b_ref[...]
    v = jnp.where(v > 0.0, v, jnp.exp(v) - 1.0)
    mask = lax.broadcasted_iota(jnp.int32, v.shape, 1) < _NCLS
    vm = jnp.where(mask, v, -jnp.inf)
    m = jnp.max(vm, axis=1, keepdims=True)
    sm = jnp.sum(jnp.where(mask, jnp.exp(vm - m), 0.0), axis=1, keepdims=True)
    out_ref[...] = v - (jnp.log(sm) + m)


def _epilogue(out2, den16, b2p):
    return pl.pallas_call(
        _epi_body,
        grid=(_N // _BLK,),
        in_specs=[
            pl.BlockSpec((_BLK, 128), lambda i: (i, 0)),
            pl.BlockSpec((_BLK, _HEADS), lambda i: (i, 0)),
            pl.BlockSpec((1, 128), lambda i: (0, 0)),
        ],
        out_specs=pl.BlockSpec((_BLK, 128), lambda i: (i, 0)),
        out_shape=jax.ShapeDtypeStruct((_N, 128), _f32),
    )(out2, den16, b2p)


def kernel(x, edge_index, W1, a_src1, a_dst1, b1, W2, a_src2, a_dst2, b2):
    src = edge_index[0]
    dst = edge_index[1]

    h, als, ald = _layer1_dense(x, W1, a_src1, a_dst1)
    g8 = jnp.maximum(jnp.max(als, axis=0) + jnp.max(ald, axis=0), 0.0)
    g16 = jnp.pad(g8, (0, 8))

    src_a = jnp.concatenate(
        [src, jnp.zeros((_EPAD - _E,), jnp.int32)]).reshape(_CHA_TOT, _CA)
    dst_a = jnp.concatenate(
        [dst, jnp.full((_EPAD - _E,), _N, jnp.int32)]).reshape(_CHA_TOT, _CA)
    src_b = src_a.reshape(_CHB_TOT, _CB)
    dst_b = dst_a.reshape(_CHB_TOT, _CB)

    als_p = jnp.pad(als, ((0, _NPAD - _N), (0, 128 - _HEADS)))
    ald_p = jnp.pad(ald, ((0, _NPAD - _N), (0, 128 - _HEADS)))
    w1, den1p = _attn_kernel()(src_a, dst_a, als_p, ald_p, g16)
    den1 = (den1p[0] + den1p[1])[:_N, :_HEADS]

    hp = [jnp.pad(h[:, 128 * i:128 * (i + 1)], ((0, _NPAD - _N), (0, 0)))
          for i in range(4)]
    o0, o1, o2, o3 = _agg1_kernel()(src_b, dst_b, w1,
                                    hp[0], hp[1], hp[2], hp[3])
    h1s = jnp.concatenate([o0[:_N], o1[:_N], o2[:_N], o3[:_N]], axis=1)

    b1_2d = b1.reshape(1, _HEADS * _HID)
    W2p = jnp.pad(W2, ((0, 0), (0, 128 - _NCLS)))
    A2 = jnp.zeros((128, 128), _f32)
    A2 = A2.at[:_NCLS, 0].set(a_src2[0])
    A2 = A2.at[:_NCLS, 1].set(a_dst2[0])
    h2, al2 = _dense2(h1s, den1, b1_2d, W2p, A2)

    as2 = al2[:, 0]
    ad2 = al2[:, 1]
    g2 = jnp.maximum(jnp.max(as2) + jnp.max(ad2), 0.0)
    g16b = jnp.full((16,), g2, _f32)
    als2_p = jnp.zeros((_NPAD, 128), _f32).at[:_N, 0].set(as2)
    ald2_p = jnp.zeros((_NPAD, 128), _f32).at[:_N, 0].set(ad2)
    w2 = _attn2_kernel()(src_a, dst_a, als2_p, ald2_p, g16b)
    if isinstance(w2, (list, tuple)):
        w2 = w2[0]

    h2p = jnp.pad(h2, ((0, _NPAD - _N), (0, 0))).at[:_N, _NCLS].set(1.0)
    o2p = _agg2_kernel()(src_b, dst_b, w2, h2p)
    if isinstance(o2p, (list, tuple)):
        o2p = o2p[0]
    out2 = (o2p[0] + o2p[1])[:_N]
    den2 = out2[:, _NCLS]

    den16 = jnp.pad(den2[:, None], ((0, 0), (0, _HEADS - 1)))
    b2p = jnp.pad(b2, (0, 128 - _NCLS)).reshape(1, 128)
    out = _epilogue(out2, den16, b2p)
    return out[:, :_NCLS]


# speedup vs baseline: 9.1747x; 1.2045x over previous
import functools

import jax
import jax.numpy as jnp
from jax import lax
from jax.experimental import pallas as pl
from jax.experimental.pallas import tpu as pltpu
from jax.experimental.pallas import tpu_sc as plsc

_N = 10000
_E = 320000
_DIN = 128
_HID = 64
_HEADS = 8
_NCLS = 40

_NC = 2
_NS = 16
_NPAD = 10240
_EPAD = 327680
_RH = 5120
_RACC = 5376
_ROOB = 5200
_BCH = 8

_CA = 64
_CHA_TOT = _EPAD // _CA
_CHA_W = _CHA_TOT // (_NC * _NS)
_CWA = _BCH * _CA * 16

_CB = 128
_CHB_TOT = _EPAD // _CB
_CHB_W = _CHB_TOT // (_NC * _NS)
_CHB_S = _CHB_TOT // _NS
_CWB = _BCH * _CB * 16

_ZRA = _RACC // _NS
_WRH = _RH // _NS

_f32 = jnp.float32


@functools.cache
def _sc_mesh():
    return plsc.VectorSubcoreMesh(core_axis_name="c", subcore_axis_name="s")


def _zero16():
    return jnp.zeros((16,), _f32)


def _fill_zb(zb):
    def _zrow(i, _):
        for k in range(8):
            zb[i, pl.ds(k * 16, 16)] = _zero16()
        return 0
    lax.fori_loop(0, 16, _zrow, 0)


def _zero_acc(zb, acc_sh, s):
    def _zcp(t, _):
        pltpu.sync_copy(zb, acc_sh.at[pl.ds(s * _ZRA + t * 16, 16), :])
        return 0
    lax.fori_loop(0, _ZRA // 16, _zcp, 0)


def _clamp_idx(didx, didx2, nrows, base, groups=8):
    def _row(r, _):
        for k in range(groups):
            sl = pl.ds(k * 16, 16)
            v = didx[r, sl] - base
            oob = (v < 0) | (v >= _RH)
            didx2[r, sl] = jnp.where(oob, _ROOB, v)
        return 0
    lax.fori_loop(0, nrows, _row, 0)


def _attn_gather_sweep(als, ald, sidx, didx, asv, asv2, adv, adv2,
                       sems, semd, g, wflat, edge_extra):
    a_bufs = (asv, asv2)
    d_bufs = (adv, adv2)
    hs = [None] * _BCH
    hd = [None] * _BCH
    hs[0] = pltpu.async_copy(als.at[sidx.at[0]], asv, sems)
    hd[0] = pltpu.async_copy(ald.at[didx.at[0]], adv, semd)
    for b in range(_BCH):
        hs[b].wait()
        hd[b].wait()
        if b + 1 < _BCH:
            hs[b + 1] = pltpu.async_copy(
                als.at[sidx.at[b + 1]], a_bufs[(b + 1) % 2], sems)
            hd[b + 1] = pltpu.async_copy(
                ald.at[didx.at[b + 1]], d_bufs[(b + 1) % 2], semd)
        ab = a_bufs[b % 2]
        db = d_bufs[b % 2]

        def _edge(j, ___, ab=ab, db=db, b=b):
            v = ab[j, pl.ds(0, 16)] + db[j, pl.ds(0, 16)]
            e = jnp.where(v >= 0.0, v, 0.2 * v)
            w = jnp.exp(e - g)
            wflat[pl.ds((b * _CA + j) * 16, 16)] = w
            edge_extra(j, w)
            return 0
        lax.fori_loop(0, _CA, _edge, 0)
        yield b


def _attn_body(src2, dst2, als, ald, g16, w_out, den_out,
               sidx, didx, didx2, asv, asv2, adv, adv2, wflat, wpad, gv, zb,
               den_sh, sems, semd):
    c = lax.axis_index("c")
    s = lax.axis_index("s")
    wid = s * _NC + c
    pltpu.sync_copy(g16, gv)
    _fill_zb(zb)

    def _zpad(j, _):
        for k in range(8):
            wpad[j, pl.ds(k * 16, 16)] = _zero16()
        return 0
    lax.fori_loop(0, _CA, _zpad, 0)

    g = gv[...]

    for half in range(2):
        base = half * _RH
        _zero_acc(zb, den_sh, s)
        plsc.subcore_barrier()

        def _batch(t, _, half=half):
            ch0 = wid * _CHA_W + t * _BCH
            pltpu.sync_copy(dst2.at[pl.ds(ch0, _BCH), :], didx)
            _clamp_idx(didx, didx2, _BCH, base, groups=_CA // 16)
            if half == 0:
                pltpu.sync_copy(src2.at[pl.ds(ch0, _BCH), :], sidx)
            else:
                pltpu.sync_copy(w_out.at[pl.ds(ch0 * _CA * 16, _CWA)], wflat)

            if half == 0:
                def _wp(j, w):
                    wpad[j, pl.ds(0, 16)] = w
                for b in _attn_gather_sweep(als, ald, sidx, didx,
                                            asv, asv2, adv, adv2,
                                            sems, semd, g, wflat, _wp):
                    pltpu.sync_copy(wpad, den_sh.at[didx2.at[b]], add=True)
                pltpu.sync_copy(wflat, w_out.at[pl.ds(ch0 * _CA * 16, _CWA)])
            else:
                def _chunk(b, __):
                    def _edge(j, ___):
                        wpad[j, pl.ds(0, 16)] = \
                            wflat[pl.ds((b * _CA + j) * 16, 16)]
                        return 0
                    lax.fori_loop(0, _CA, _edge, 0)
                    pltpu.sync_copy(wpad, den_sh.at[didx2.at[b]], add=True)
                    return 0
                lax.fori_loop(0, _BCH, _chunk, 0)
            return 0
        lax.fori_loop(0, _CHA_W // _BCH, _batch, 0)

        plsc.subcore_barrier()
        pltpu.sync_copy(den_sh.at[pl.ds(s * _WRH, _WRH), :],
                        den_out.at[c, pl.ds(base + s * _WRH, _WRH), :])
        plsc.subcore_barrier()


@functools.cache
def _attn_kernel():
    return pl.kernel(
        _attn_body,
        mesh=_sc_mesh(),
        out_type=[jax.ShapeDtypeStruct((_EPAD * 16,), _f32),
                  jax.ShapeDtypeStruct((_NC, _NPAD, 128), _f32)],
        scratch_types=[pltpu.VMEM((_BCH, _CA), jnp.int32),
                       pltpu.VMEM((_BCH, _CA), jnp.int32),
                       pltpu.VMEM((_BCH, _CA), jnp.int32),
                       pltpu.VMEM((_CA, 128), _f32),
                       pltpu.VMEM((_CA, 128), _f32),
                       pltpu.VMEM((_CA, 128), _f32),
                       pltpu.VMEM((_CA, 128), _f32),
                       pltpu.VMEM((_CWA,), _f32),
                       pltpu.VMEM((_CA, 128), _f32),
                       pltpu.VMEM((16,), _f32),
                       pltpu.VMEM((16, 128), _f32),
                       pltpu.VMEM_SHARED((_RACC, 128), _f32),
                       pltpu.SemaphoreType.DMA,
                       pltpu.SemaphoreType.DMA],
    )


def _attn2_body(src2, dst2, als, ald, g16, w_out,
                sidx, didx, asv, asv2, adv, adv2, wflat, gv, sems, semd):
    c = lax.axis_index("c")
    s = lax.axis_index("s")
    wid = s * _NC + c
    pltpu.sync_copy(g16, gv)
    g = gv[...]

    def _noop(j, w):
        pass

    def _batch(t, _):
        ch0 = wid * _CHA_W + t * _BCH
        pltpu.sync_copy(src2.at[pl.ds(ch0, _BCH), :], sidx)
        pltpu.sync_copy(dst2.at[pl.ds(ch0, _BCH), :], didx)
        for _b in _attn_gather_sweep(als, ald, sidx, didx, asv, asv2,
                                     adv, adv2, sems, semd, g, wflat, _noop):
            pass
        pltpu.sync_copy(wflat, w_out.at[pl.ds(ch0 * _CA * 16, _CWA)])
        return 0
    lax.fori_loop(0, _CHA_W // _BCH, _batch, 0)


@functools.cache
def _attn2_kernel():
    return pl.kernel(
        _attn2_body,
        mesh=_sc_mesh(),
        out_type=[jax.ShapeDtypeStruct((_EPAD * 16,), _f32)],
        scratch_types=[pltpu.VMEM((_BCH, _CA), jnp.int32),
                       pltpu.VMEM((_BCH, _CA), jnp.int32),
                       pltpu.VMEM((_CA, 128), _f32),
                       pltpu.VMEM((_CA, 128), _f32),
                       pltpu.VMEM((_CA, 128), _f32),
                       pltpu.VMEM((_CA, 128), _f32),
                       pltpu.VMEM((_CWA,), _f32),
                       pltpu.VMEM((16,), _f32),
                       pltpu.SemaphoreType.DMA,
                       pltpu.SemaphoreType.DMA],
    )


def _scaled_scatter(hp, acc_sh, sidx, didx2, rows_a, rows_b, sem, scale_fn):
    bufs = (rows_a, rows_b)
    handles = [None] * _BCH
    handles[0] = pltpu.async_copy(hp.at[sidx.at[0]], bufs[0], sem)
    for b in range(_BCH):
        handles[b].wait()
        if b + 1 < _BCH:
            handles[b + 1] = pltpu.async_copy(
                hp.at[sidx.at[b + 1]], bufs[(b + 1) % 2], sem)
        rbuf = bufs[b % 2]

        def _edge(j, ___, b=b, rbuf=rbuf):
            scale_fn(rbuf, b, j)
            return 0
        lax.fori_loop(0, _CB, _edge, 0)
        pltpu.sync_copy(rbuf, acc_sh.at[didx2.at[b]], add=True)


def _agg1_body(src2, dst2, w_hbm, h0, h1, h2, h3, o0, o1, o2, o3,
               sidx, didx, didx2, wflat, rows_a, rows_b, zb, acc_sh, sem):
    c = lax.axis_index("c")
    s = lax.axis_index("s")
    _fill_zb(zb)

    for p in range(4):
        hp = (h0, h1, h2, h3)[p]
        op = (o0, o1, o2, o3)[p]

        @pl.when(c == p // 2)
        def _pass(hp=hp, op=op, p=p):
            def _scale(rbuf, b, j, p=p):
                wrow = wflat[pl.ds((b * _CB + j) * 16, 16)]
                w0 = wrow[2 * p]
                w1 = wrow[2 * p + 1]
                for k in range(8):
                    sl = pl.ds(k * 16, 16)
                    ww = w0 if k < 4 else w1
                    rbuf[j, sl] = rbuf[j, sl] * ww

            for half in range(2):
                base = half * _RH
                _zero_acc(zb, acc_sh, s)
                plsc.subcore_barrier()

                def _batch(t, _):
                    ch0 = s * _CHB_S + t * _BCH
                    pltpu.sync_copy(src2.at[pl.ds(ch0, _BCH), :], sidx)
                    pltpu.sync_copy(dst2.at[pl.ds(ch0, _BCH), :], didx)
                    pltpu.sync_copy(
                        w_hbm.at[pl.ds(ch0 * _CB * 16, _CWB)], wflat)
                    _clamp_idx(didx, didx2, _BCH, base)
                    _scaled_scatter(hp, acc_sh, sidx, didx2,
                                    rows_a, rows_b, sem, _scale)
                    return 0
                lax.fori_loop(0, _CHB_S // _BCH, _batch, 0)

                plsc.subcore_barrier()
                pltpu.sync_copy(acc_sh.at[pl.ds(s * _WRH, _WRH), :],
                                op.at[pl.ds(base + s * _WRH, _WRH), :])
                plsc.subcore_barrier()


@functools.cache
def _agg1_kernel():
    return pl.kernel(
        _agg1_body,
        mesh=_sc_mesh(),
        out_type=[jax.ShapeDtypeStruct((_NPAD, 128), _f32)] * 4,
        scratch_types=[pltpu.VMEM((_BCH, _CB), jnp.int32),
                       pltpu.VMEM((_BCH, _CB), jnp.int32),
                       pltpu.VMEM((_BCH, _CB), jnp.int32),
                       pltpu.VMEM((_CWB,), _f32),
                       pltpu.VMEM((_CB, 128), _f32),
                       pltpu.VMEM((_CB, 128), _f32),
                       pltpu.VMEM((16, 128), _f32),
                       pltpu.VMEM_SHARED((_RACC, 128), _f32),
                       pltpu.SemaphoreType.DMA],
    )


def _agg2_body(src2, dst2, w_hbm, h2p, o_out,
               sidx, didx, didx2, wflat, rows_a, rows_b, zb, acc_sh, sem):
    c = lax.axis_index("c")
    s = lax.axis_index("s")
    wid = s * _NC + c
    _fill_zb(zb)

    def _scale(rbuf, b, j):
        wrow = wflat[pl.ds((b * _CB + j) * 16, 16)]
        w0 = wrow[0]
        for k in range(8):
            sl = pl.ds(k * 16, 16)
            rbuf[j, sl] = rbuf[j, sl] * w0

    for half in range(2):
        base = half * _RH
        _zero_acc(zb, acc_sh, s)
        plsc.subcore_barrier()

        def _batch(t, _):
            ch0 = wid * _CHB_W + t * _BCH
            pltpu.sync_copy(src2.at[pl.ds(ch0, _BCH), :], sidx)
            pltpu.sync_copy(dst2.at[pl.ds(ch0, _BCH), :], didx)
            pltpu.sync_copy(w_hbm.at[pl.ds(ch0 * _CB * 16, _CWB)], wflat)
            _clamp_idx(didx, didx2, _BCH, base)
            _scaled_scatter(h2p, acc_sh, sidx, didx2,
                            rows_a, rows_b, sem, _scale)
            return 0
        lax.fori_loop(0, _CHB_W // _BCH, _batch, 0)

        plsc.subcore_barrier()
        pltpu.sync_copy(acc_sh.at[pl.ds(s * _WRH, _WRH), :],
                        o_out.at[c, pl.ds(base + s * _WRH, _WRH), :])
        plsc.subcore_barrier()


@functools.cache
def _agg2_kernel():
    return pl.kernel(
        _agg2_body,
        mesh=_sc_mesh(),
        out_type=[jax.ShapeDtypeStruct((_NC, _NPAD, 128), _f32)],
        scratch_types=[pltpu.VMEM((_BCH, _CB), jnp.int32),
                       pltpu.VMEM((_BCH, _CB), jnp.int32),
                       pltpu.VMEM((_BCH, _CB), jnp.int32),
                       pltpu.VMEM((_CWB,), _f32),
                       pltpu.VMEM((_CB, 128), _f32),
                       pltpu.VMEM((_CB, 128), _f32),
                       pltpu.VMEM((16, 128), _f32),
                       pltpu.VMEM_SHARED((_RACC, 128), _f32),
                       pltpu.SemaphoreType.DMA],
    )


_BLK = 1000


def _l1_body(x_ref, w_ref, asrc_ref, adst_ref, h_ref, as_ref, ad_ref):
    h = jnp.dot(x_ref[...], w_ref[...], preferred_element_type=_f32)
    h_ref[...] = h
    hh = h.reshape(h.shape[0], _HEADS, _HID)
    as_ref[...] = (hh * asrc_ref[...][None]).sum(-1)
    ad_ref[...] = (hh * adst_ref[...][None]).sum(-1)


def _layer1_dense(x, W1, a_src1, a_dst1):
    return pl.pallas_call(
        _l1_body,
        grid=(_N // _BLK,),
        in_specs=[
            pl.BlockSpec((_BLK, _DIN), lambda i: (i, 0)),
            pl.BlockSpec((_DIN, _HEADS * _HID), lambda i: (0, 0)),
            pl.BlockSpec((_HEADS, _HID), lambda i: (0, 0)),
            pl.BlockSpec((_HEADS, _HID), lambda i: (0, 0)),
        ],
        out_specs=[
            pl.BlockSpec((_BLK, _HEADS * _HID), lambda i: (i, 0)),
            pl.BlockSpec((_BLK, _HEADS), lambda i: (i, 0)),
            pl.BlockSpec((_BLK, _HEADS), lambda i: (i, 0)),
        ],
        out_shape=[
            jax.ShapeDtypeStruct((_N, _HEADS * _HID), _f32),
            jax.ShapeDtypeStruct((_N, _HEADS), _f32),
            jax.ShapeDtypeStruct((_N, _HEADS), _f32),
        ],
    )(x, W1, a_src1, a_dst1)


def _d2_body(h_ref, den_ref, b_ref, w_ref, a_ref, h2_ref, al_ref):
    den = den_ref[...][:, :, None]
    hn = h_ref[...].reshape(-1, _HEADS, _HID) / (den + 1e-16)
    hn = hn.reshape(h_ref.shape[0], _HEADS * _HID) + b_ref[...]
    hn = jnp.where(hn > 0.0, hn, jnp.exp(hn) - 1.0)
    h2 = jnp.dot(hn, w_ref[...], preferred_element_type=_f32)
    h2_ref[...] = h2
    al_ref[...] = jnp.dot(h2, a_ref[...], preferred_element_type=_f32)


def _dense2(h1s, den1, b1_2d, W2p, A2):
    return pl.pallas_call(
        _d2_body,
        grid=(_N // _BLK,),
        in_specs=[
            pl.BlockSpec((_BLK, _HEADS * _HID), lambda i: (i, 0)),
            pl.BlockSpec((_BLK, _HEADS), lambda i: (i, 0)),
            pl.BlockSpec((1, _HEADS * _HID), lambda i: (0, 0)),
            pl.BlockSpec((_HEADS * _HID, 128), lambda i: (0, 0)),
            pl.BlockSpec((128, 128), lambda i: (0, 0)),
        ],
        out_specs=[
            pl.BlockSpec((_BLK, 128), lambda i: (i, 0)),
            pl.BlockSpec((_BLK, 128), lambda i: (i, 0)),
        ],
        out_shape=[
            jax.ShapeDtypeStruct((_N, 128), _f32),
            jax.ShapeDtypeStruct((_N, 128), _f32),
        ],
    )(h1s, den1, b1_2d, W2p, A2)


def _epi_body(o_ref, d_ref, b_ref, out_ref):
    d = d_ref[...][:, 0:1]
    v = o_ref[...] / (d + 1e-16) + b_ref[...]
    v = jnp.where(v > 0.0, v, jnp.exp(v) - 1.0)
    mask = lax.broadcasted_iota(jnp.int32, v.shape, 1) < _NCLS
    vm = jnp.where(mask, v, -jnp.inf)
    m = jnp.max(vm, axis=1, keepdims=True)
    sm = jnp.sum(jnp.where(mask, jnp.exp(vm - m), 0.0), axis=1, keepdims=True)
    out_ref[...] = v - (jnp.log(sm) + m)


def _epilogue(out2, den16, b2p):
    return pl.pallas_call(
        _epi_body,
        grid=(_N // _BLK,),
        in_specs=[
            pl.BlockSpec((_BLK, 128), lambda i: (i, 0)),
            pl.BlockSpec((_BLK, _HEADS), lambda i: (i, 0)),
            pl.BlockSpec((1, 128), lambda i: (0, 0)),
        ],
        out_specs=pl.BlockSpec((_BLK, 128), lambda i: (i, 0)),
        out_shape=jax.ShapeDtypeStruct((_N, 128), _f32),
    )(out2, den16, b2p)


def kernel(x, edge_index, W1, a_src1, a_dst1, b1, W2, a_src2, a_dst2, b2):
    src = edge_index[0]
    dst = edge_index[1]

    h, als, ald = _layer1_dense(x, W1, a_src1, a_dst1)
    g8 = jnp.maximum(jnp.max(als, axis=0) + jnp.max(ald, axis=0), 0.0)
    g16 = jnp.pad(g8, (0, 8))

    src_a = jnp.concatenate(
        [src, jnp.zeros((_EPAD - _E,), jnp.int32)]).reshape(_CHA_TOT, _CA)
    dst_a = jnp.concatenate(
        [dst, jnp.full((_EPAD - _E,), _N, jnp.int32)]).reshape(_CHA_TOT, _CA)
    src_b = src_a.reshape(_CHB_TOT, _CB)
    dst_b = dst_a.reshape(_CHB_TOT, _CB)

    als_p = jnp.pad(als, ((0, _NPAD - _N), (0, 128 - _HEADS)))
    ald_p = jnp.pad(ald, ((0, _NPAD - _N), (0, 128 - _HEADS)))
    w1, den1p = _attn_kernel()(src_a, dst_a, als_p, ald_p, g16)
    den1 = (den1p[0] + den1p[1])[:_N, :_HEADS]

    hp = [jnp.pad(h[:, 128 * i:128 * (i + 1)], ((0, _NPAD - _N), (0, 0)))
          for i in range(4)]
    o0, o1, o2, o3 = _agg1_kernel()(src_b, dst_b, w1,
                                    hp[0], hp[1], hp[2], hp[3])
    h1s = jnp.concatenate([o0[:_N], o1[:_N], o2[:_N], o3[:_N]], axis=1)

    b1_2d = b1.reshape(1, _HEADS * _HID)
    W2p = jnp.pad(W2, ((0, 0), (0, 128 - _NCLS)))
    A2 = jnp.zeros((128, 128), _f32)
    A2 = A2.at[:_NCLS, 0].set(a_src2[0])
    A2 = A2.at[:_NCLS, 1].set(a_dst2[0])
    h2, al2 = _dense2(h1s, den1, b1_2d, W2p, A2)

    as2 = al2[:, 0]
    ad2 = al2[:, 1]
    g2 = jnp.maximum(jnp.max(as2) + jnp.max(ad2), 0.0)
    g16b = jnp.full((16,), g2, _f32)
    als2_p = jnp.zeros((_NPAD, 128), _f32).at[:_N, 0].set(as2)
    ald2_p = jnp.zeros((_NPAD, 128), _f32).at[:_N, 0].set(ad2)
    w2 = _attn2_kernel()(src_a, dst_a, als2_p, ald2_p, g16b)
    if isinstance(w2, (list, tuple)):
        w2 = w2[0]

    h2p = jnp.pad(h2, ((0, _NPAD - _N), (0, 0))).at[:_N, _NCLS].set(1.0)
    o2p = _agg2_kernel()(src_b, dst_b, w2, h2p)
    if isinstance(o2p, (list, tuple)):
        o2p = o2p[0]
    out2 = (o2p[0] + o2p[1])[:_N]
    den2 = out2[:, _NCLS]

    den16 = jnp.pad(den2[:, None], ((0, 0), (0, _HEADS - 1)))
    b2p = jnp.pad(b2, (0, 128 - _NCLS)).reshape(1, 128)
    out = _epilogue(out2, den16, b2p)
    return out[:, :_NCLS]
